# baseline (speedup 1.0000x reference)
import functools

import jax
import jax.numpy as jnp
from jax import lax
from jax.experimental import pallas as pl
from jax.experimental.pallas import tpu as pltpu
from jax.experimental.pallas import tpu_sc as plsc

V = 50257
NU = 10000
NI = 20000
NLAB = NU + NI
D = 768
KC = 16
B = 8
S = 2048
NTOK = B * S
NC, NS, L = 2, 16, 16
NW = NC * NS
TPW = NTOK // NW
C = 48
NCH = (TPW + C - 1) // C
LLEN = NCH * C
EPAD = 64
DUMP_E = NTOK
ANB = 2 * KC
DUMP_A = B * ANB

_mesh = plsc.VectorSubcoreMesh(core_axis_name="c", subcore_axis_name="s")


@functools.partial(
    pl.kernel,
    out_type=(
        jax.ShapeDtypeStruct((NTOK, D), jnp.float32),
        jax.ShapeDtypeStruct((NTOK,), jnp.int32),
        jax.ShapeDtypeStruct((DUMP_A + KC, D), jnp.float32),
    ),
    mesh=_mesh,
    compiler_params=pltpu.CompilerParams(needs_layout_passes=False),
    scratch_types=[
        pltpu.VMEM((TPW,), jnp.int32),
        pltpu.VMEM((TPW,), jnp.int32),
        pltpu.VMEM((LLEN,), jnp.int32),
        pltpu.VMEM((LLEN,), jnp.int32),
        pltpu.VMEM((LLEN,), jnp.int32),
        pltpu.VMEM((LLEN,), jnp.int32),
        pltpu.VMEM((LLEN,), jnp.int32),
        pltpu.VMEM((LLEN,), jnp.int32),
        pltpu.VMEM((C,), jnp.int32),
        pltpu.VMEM((C,), jnp.int32),
        pltpu.VMEM((C,), jnp.int32),
        pltpu.VMEM((C, D), jnp.float32),
        pltpu.VMEM((C, D), jnp.float32),
        pltpu.VMEM((S,), jnp.int32),
        pltpu.VMEM((NLAB,), jnp.int32),
        pltpu.VMEM((KC,), jnp.int32),
        pltpu.VMEM((KC,), jnp.int32),
        pltpu.SemaphoreType.DMA,
        pltpu.SemaphoreType.DMA,
        pltpu.SemaphoreType.DMA,
    ],
)
def _sc_embed(ids_hbm, wte, uemb, iemb, cen01, lab01,
              emb_o, nl_o, an_o,
              ids_v, nl_v, r0, r1, r2, p0, p1, p2,
              rid_stage, pos_stage_a, pos_stage_b, rows_a, rows_b,
              ids_row, lab_v, pres_v, dest_stage, sem_g, sem_sa, sem_sb):
    wid = lax.axis_index("s") * NC + lax.axis_index("c")
    base = wid * TPW
    iota = lax.iota(jnp.int32, L)

    pltpu.sync_copy(ids_hbm.at[pl.ds(base, TPW)], ids_v)

    zi = jnp.zeros((L,), jnp.int32)

    def build_body(k, carry):
        ids = ids_v[pl.ds(k * L, L)]
        is_voc = ids < V
        is_usr = jnp.logical_and(ids >= V, ids < V + NU)
        is_itm = ids >= V + NU
        nl_v[pl.ds(k * L, L)] = jnp.where(is_voc, 0, 1).astype(jnp.int32)
        pos = base + k * L + iota
        rows = (ids, ids - V, ids - V - NU)
        masks = (is_voc, is_usr, is_itm)
        rlists = (r0, r1, r2)
        plists = (p0, p1, p2)
        out = []
        for t in range(3):
            mi = jnp.where(masks[t], 1, 0)
            cs = plsc.cumsum(mi)
            dest = carry[t] + cs - 1
            plsc.store_scatter(rlists[t], [dest], rows[t], mask=masks[t])
            plsc.store_scatter(plists[t], [dest], pos, mask=masks[t])
            out.append(carry[t] + plsc.all_reduce_population_count(masks[t]))
        return tuple(out)

    zv = jnp.zeros((L,), jnp.int32)
    cntv = lax.fori_loop(0, TPW // L, build_body, (zv, zv, zv))
    cnts = tuple(jnp.max(c, axis=0) for c in cntv)

    pltpu.sync_copy(nl_v, nl_o.at[pl.ds(base, TPW)])

    for t in range(3):
        cnt = cnts[t]
        rlist = (r0, r1, r2)[t]
        plist = (p0, p1, p2)[t]
        rspl = plsc.load_gather(rlist, [zi])
        pspl = plsc.load_gather(plist, [zi])
        lim = ((cnt + C - 1) // C) * C

        def fill_body(k, _, rlist=rlist, plist=plist, rspl=rspl, pspl=pspl,
                      cnt=cnt, lim=lim):
            idx = k * L + iota
            m = jnp.logical_and(idx >= cnt, idx < lim)
            rv = rlist[pl.ds(k * L, L)]
            pv = plist[pl.ds(k * L, L)]
            rlist[pl.ds(k * L, L)] = jnp.where(m, rspl, rv)
            plist[pl.ds(k * L, L)] = jnp.where(m, pspl, pv)
            return 0

        lax.fori_loop(0, LLEN // L, fill_body, 0)

    def _drain_scatter(sem):
        pltpu.make_async_copy(emb_o.at[pl.ds(0, C)], rows_a, sem).wait()

    for t, table in enumerate((wte, uemb, iemb)):
        cnt = cnts[t]
        rlist = (r0, r1, r2)[t]
        plist = (p0, p1, p2)[t]

        def pair_body(g, _, rlist=rlist, plist=plist, table=table, cnt=cnt):
            for par in range(2):
                buf = (rows_a, rows_b)[par]
                pstage = (pos_stage_a, pos_stage_b)[par]
                sem_s = (sem_sa, sem_sb)[par]
                ch = g * 2 + par

                @pl.when(ch * C < cnt)
                def _(ch=ch, buf=buf, pstage=pstage, sem_s=sem_s):
                    @pl.when(ch > 1)
                    def _():
                        _drain_scatter(sem_s)

                    for j in range(C // L):
                        rid_stage[pl.ds(j * L, L)] = rlist[pl.ds(ch * C + j * L, L)]
                        pstage[pl.ds(j * L, L)] = plist[pl.ds(ch * C + j * L, L)]
                    pltpu.async_copy(table.at[rid_stage], buf, sem_g).wait()
                    pltpu.async_copy(buf, emb_o.at[pstage], sem_s)
            return 0

        lax.fori_loop(0, (NCH + 1) // 2, pair_body, 0)

        @pl.when(cnt > 0)
        def _():
            _drain_scatter(sem_sa)

        @pl.when(cnt > C)
        def _():
            _drain_scatter(sem_sb)

    @pl.when(wid < B * 2)
    def _anode():
        b = wid // 2
        lsel = wid - 2 * b
        pltpu.sync_copy(ids_hbm.at[pl.ds(b * S, S)], ids_row)
        pltpu.sync_copy(lab01.at[pl.ds(lsel * NLAB, NLAB)], lab_v)

        pres_v[...] = jnp.zeros((KC,), jnp.int32)
        ones = jnp.ones((L,), jnp.int32)

        def pres_body(k, _):
            ids = ids_row[pl.ds(k * L, L)]
            m = ids >= V
            seq = jnp.where(m, ids - V, 0)
            lab = plsc.load_gather(lab_v, [seq])
            plsc.store_scatter(pres_v, [lab], ones, mask=m)
            return 0

        lax.fori_loop(0, S // L, pres_body, 0)

        pi = pres_v[...]
        rank = plsc.cumsum(pi) - pi
        abase = b * ANB + lsel * KC
        dest_stage[...] = jnp.where(pi > 0, abase + rank, DUMP_A + iota)

        zf = jnp.zeros((L,), jnp.float32)

        def zero_body(r, _):
            for cidx in range(D // L):
                rows_a[r, pl.ds(cidx * L, L)] = zf
            return 0

        lax.fori_loop(0, KC, zero_body, 0)
        pltpu.sync_copy(rows_a.at[pl.ds(0, KC)], an_o.at[pl.ds(abase, KC)])

        crows = rows_b.at[pl.ds(0, KC)]
        pltpu.sync_copy(cen01.at[pl.ds(lsel * KC, KC)], crows)
        pltpu.async_copy(crows, an_o.at[dest_stage], sem_g).wait()


def kernel(input_ids, wte, user_emb, item_emb, a_centers0, a_centers1,
           a_labels0, a_labels1):
    ids = input_ids.reshape(-1).astype(jnp.int32)
    cen01 = jnp.concatenate([a_centers0, a_centers1], axis=0)
    lab01 = jnp.concatenate(
        [a_labels0.astype(jnp.int32), a_labels1.astype(jnp.int32)], axis=0)
    emb, nl, an = _sc_embed(ids, wte, user_emb, item_emb, cen01, lab01)
    return (emb.reshape(B, S, D),
            nl.reshape(B, S),
            an[:DUMP_A].reshape(B, ANB, D))


# speedup vs baseline: 1.0489x; 1.0489x over previous
import functools

import jax
import jax.numpy as jnp
from jax import lax
from jax.experimental import pallas as pl
from jax.experimental.pallas import tpu as pltpu
from jax.experimental.pallas import tpu_sc as plsc

V = 50257
NU = 10000
NI = 20000
NLAB = NU + NI
D = 768
KC = 16
B = 8
S = 2048
NTOK = B * S
NC, NS, L = 2, 16, 16
NW = NC * NS
TPW = NTOK // NW
C = 48
NCH = (TPW + C - 1) // C
LLEN = NCH * C
EPAD = 64
DUMP_E = NTOK
ANB = 2 * KC
DUMP_A = B * ANB

_mesh = plsc.VectorSubcoreMesh(core_axis_name="c", subcore_axis_name="s")


@functools.partial(
    pl.kernel,
    out_type=(
        jax.ShapeDtypeStruct((NTOK, D), jnp.float32),
        jax.ShapeDtypeStruct((NTOK,), jnp.int32),
        jax.ShapeDtypeStruct((DUMP_A + KC, D), jnp.float32),
    ),
    mesh=_mesh,
    compiler_params=pltpu.CompilerParams(needs_layout_passes=False),
    scratch_types=[
        pltpu.VMEM((TPW,), jnp.int32),
        pltpu.VMEM((TPW,), jnp.int32),
        pltpu.VMEM((LLEN,), jnp.int32),
        pltpu.VMEM((LLEN,), jnp.int32),
        pltpu.VMEM((LLEN,), jnp.int32),
        pltpu.VMEM((LLEN,), jnp.int32),
        pltpu.VMEM((LLEN,), jnp.int32),
        pltpu.VMEM((LLEN,), jnp.int32),
        pltpu.VMEM((C,), jnp.int32),
        pltpu.VMEM((C,), jnp.int32),
        pltpu.VMEM((C,), jnp.int32),
        pltpu.VMEM((C,), jnp.int32),
        pltpu.VMEM((C, D), jnp.float32),
        pltpu.VMEM((C, D), jnp.float32),
        pltpu.VMEM((S,), jnp.int32),
        pltpu.VMEM((NLAB,), jnp.int32),
        pltpu.VMEM((KC,), jnp.int32),
        pltpu.VMEM((KC,), jnp.int32),
        pltpu.SemaphoreType.DMA,
        pltpu.SemaphoreType.DMA,
        pltpu.SemaphoreType.DMA,
        pltpu.SemaphoreType.DMA,
    ],
)
def _sc_embed(ids_hbm, wte, uemb, iemb, cen01, lab01,
              emb_o, nl_o, an_o,
              ids_v, nl_v, r0, r1, r2, p0, p1, p2,
              rid_stage_a, rid_stage_b, pos_stage_a, pos_stage_b,
              rows_a, rows_b, ids_row, lab_v, pres_v, dest_stage,
              sem_ga, sem_gb, sem_sa, sem_sb):
    wid = lax.axis_index("s") * NC + lax.axis_index("c")
    base = wid * TPW
    iota = lax.iota(jnp.int32, L)

    pltpu.sync_copy(ids_hbm.at[pl.ds(base, TPW)], ids_v)

    zi = jnp.zeros((L,), jnp.int32)

    def build_body(k, carry):
        ids = ids_v[pl.ds(k * L, L)]
        is_voc = ids < V
        is_usr = jnp.logical_and(ids >= V, ids < V + NU)
        is_itm = ids >= V + NU
        nl_v[pl.ds(k * L, L)] = jnp.where(is_voc, 0, 1).astype(jnp.int32)
        pos = base + k * L + iota
        rows = (ids, ids - V, ids - V - NU)
        masks = (is_voc, is_usr, is_itm)
        rlists = (r0, r1, r2)
        plists = (p0, p1, p2)
        out = []
        for t in range(3):
            mi = jnp.where(masks[t], 1, 0)
            cs = plsc.cumsum(mi)
            dest = carry[t] + cs - 1
            plsc.store_scatter(rlists[t], [dest], rows[t], mask=masks[t])
            plsc.store_scatter(plists[t], [dest], pos, mask=masks[t])
            out.append(carry[t] + plsc.all_reduce_population_count(masks[t]))
        return tuple(out)

    zv = jnp.zeros((L,), jnp.int32)
    cntv = lax.fori_loop(0, TPW // L, build_body, (zv, zv, zv))
    cnts = tuple(jnp.max(c, axis=0) for c in cntv)

    pltpu.sync_copy(nl_v, nl_o.at[pl.ds(base, TPW)])

    for t in range(3):
        cnt = cnts[t]
        rlist = (r0, r1, r2)[t]
        plist = (p0, p1, p2)[t]
        rspl = plsc.load_gather(rlist, [zi])
        pspl = plsc.load_gather(plist, [zi])
        lim = ((cnt + C - 1) // C) * C

        def fill_body(k, _, rlist=rlist, plist=plist, rspl=rspl, pspl=pspl,
                      cnt=cnt, lim=lim):
            idx = k * L + iota
            m = jnp.logical_and(idx >= cnt, idx < lim)
            rv = rlist[pl.ds(k * L, L)]
            pv = plist[pl.ds(k * L, L)]
            rlist[pl.ds(k * L, L)] = jnp.where(m, rspl, rv)
            plist[pl.ds(k * L, L)] = jnp.where(m, pspl, pv)
            return 0

        lax.fori_loop(0, LLEN // L, fill_body, 0)

    def _drain(sem):
        pltpu.make_async_copy(emb_o.at[pl.ds(0, C)], rows_a, sem).wait()

    for t, table in enumerate((wte, uemb, iemb)):
        cnt = cnts[t]
        nch = (cnt + C - 1) // C
        rlist = (r0, r1, r2)[t]
        plist = (p0, p1, p2)[t]

        def pair_body(g, _, rlist=rlist, plist=plist, table=table, cnt=cnt):
            for par in range(2):
                buf = (rows_a, rows_b)[par]
                obuf = (rows_a, rows_b)[1 - par]
                ridst = (rid_stage_a, rid_stage_b)[par]
                pstage = (pos_stage_a, pos_stage_b)[par]
                opstage = (pos_stage_a, pos_stage_b)[1 - par]
                sem_s = (sem_sa, sem_sb)[par]
                sem_g = (sem_ga, sem_gb)[par]
                osem_g = (sem_ga, sem_gb)[1 - par]
                osem_s = (sem_sa, sem_sb)[1 - par]
                ch = g * 2 + par

                @pl.when(ch * C < cnt)
                def _(ch=ch, buf=buf, obuf=obuf, ridst=ridst, pstage=pstage,
                      opstage=opstage, sem_s=sem_s, sem_g=sem_g,
                      osem_g=osem_g, osem_s=osem_s):
                    @pl.when(ch > 1)
                    def _():
                        _drain(sem_s)

                    for j in range(C // L):
                        ridst[pl.ds(j * L, L)] = rlist[pl.ds(ch * C + j * L, L)]
                        pstage[pl.ds(j * L, L)] = plist[pl.ds(ch * C + j * L, L)]
                    pltpu.async_copy(table.at[ridst], buf, sem_g)

                    @pl.when(ch > 0)
                    def _():
                        _drain(osem_g)
                        pltpu.async_copy(obuf, emb_o.at[opstage], osem_s)
            return 0

        lax.fori_loop(0, (NCH + 1) // 2, pair_body, 0)

        for par in range(2):
            buf = (rows_a, rows_b)[par]
            pstage = (pos_stage_a, pos_stage_b)[par]
            sem_g = (sem_ga, sem_gb)[par]
            sem_s = (sem_sa, sem_sb)[par]

            @pl.when(jnp.logical_and(cnt > 0, (nch - 1) % 2 == par))
            def _(buf=buf, pstage=pstage, sem_g=sem_g, sem_s=sem_s):
                _drain(sem_g)
                pltpu.async_copy(buf, emb_o.at[pstage], sem_s)

        @pl.when(cnt > 0)
        def _():
            _drain(sem_sa)

        @pl.when(cnt > C)
        def _():
            _drain(sem_sb)

    @pl.when(wid < B * 2)
    def _anode():
        b = wid // 2
        lsel = wid - 2 * b
        pltpu.sync_copy(ids_hbm.at[pl.ds(b * S, S)], ids_row)
        pltpu.sync_copy(lab01.at[pl.ds(lsel * NLAB, NLAB)], lab_v)

        pres_v[...] = jnp.zeros((KC,), jnp.int32)
        ones = jnp.ones((L,), jnp.int32)

        def pres_body(k, _):
            ids = ids_row[pl.ds(k * L, L)]
            m = ids >= V
            seq = jnp.where(m, ids - V, 0)
            lab = plsc.load_gather(lab_v, [seq])
            plsc.store_scatter(pres_v, [lab], ones, mask=m)
            return 0

        lax.fori_loop(0, S // L, pres_body, 0)

        pi = pres_v[...]
        rank = plsc.cumsum(pi) - pi
        abase = b * ANB + lsel * KC
        dest_stage[...] = jnp.where(pi > 0, abase + rank, DUMP_A + iota)

        zf = jnp.zeros((L,), jnp.float32)

        def zero_body(r, _):
            for cidx in range(D // L):
                rows_a[r, pl.ds(cidx * L, L)] = zf
            return 0

        lax.fori_loop(0, KC, zero_body, 0)
        pltpu.sync_copy(rows_a.at[pl.ds(0, KC)], an_o.at[pl.ds(abase, KC)])

        crows = rows_b.at[pl.ds(0, KC)]
        pltpu.sync_copy(cen01.at[pl.ds(lsel * KC, KC)], crows)
        pltpu.async_copy(crows, an_o.at[dest_stage], sem_ga).wait()


def kernel(input_ids, wte, user_emb, item_emb, a_centers0, a_centers1,
           a_labels0, a_labels1):
    ids = input_ids.reshape(-1).astype(jnp.int32)
    cen01 = jnp.concatenate([a_centers0, a_centers1], axis=0)
    lab01 = jnp.concatenate(
        [a_labels0.astype(jnp.int32), a_labels1.astype(jnp.int32)], axis=0)
    emb, nl, an = _sc_embed(ids, wte, user_emb, item_emb, cen01, lab01)
    return (emb.reshape(B, S, D),
            nl.reshape(B, S),
            an[:DUMP_A].reshape(B, ANB, D))


# speedup vs baseline: 1.0780x; 1.0278x over previous
import functools

import jax
import jax.numpy as jnp
from jax import lax
from jax.experimental import pallas as pl
from jax.experimental.pallas import tpu as pltpu
from jax.experimental.pallas import tpu_sc as plsc

V = 50257
NU = 10000
NI = 20000
NLAB = NU + NI
D = 768
KC = 16
B = 8
S = 2048
NTOK = B * S
NC, NS, L = 2, 16, 16
NW = NC * NS
TPW = NTOK // NW
C = 32
NCH = (TPW + C - 1) // C
LLEN = NCH * C
EPAD = 64
DUMP_E = NTOK
ANB = 2 * KC
DUMP_A = B * ANB

_mesh = plsc.VectorSubcoreMesh(core_axis_name="c", subcore_axis_name="s")


@functools.partial(
    pl.kernel,
    out_type=(
        jax.ShapeDtypeStruct((NTOK, D), jnp.float32),
        jax.ShapeDtypeStruct((NTOK,), jnp.int32),
        jax.ShapeDtypeStruct((DUMP_A + KC, D), jnp.float32),
    ),
    mesh=_mesh,
    compiler_params=pltpu.CompilerParams(needs_layout_passes=False),
    scratch_types=[
        pltpu.VMEM((TPW,), jnp.int32),
        pltpu.VMEM((TPW,), jnp.int32),
        pltpu.VMEM((LLEN,), jnp.int32),
        pltpu.VMEM((LLEN,), jnp.int32),
        pltpu.VMEM((LLEN,), jnp.int32),
        pltpu.VMEM((LLEN,), jnp.int32),
        pltpu.VMEM((LLEN,), jnp.int32),
        pltpu.VMEM((LLEN,), jnp.int32),
        pltpu.VMEM((C,), jnp.int32),
        pltpu.VMEM((C,), jnp.int32),
        pltpu.VMEM((C,), jnp.int32),
        pltpu.VMEM((C,), jnp.int32),
        pltpu.VMEM((C, D), jnp.float32),
        pltpu.VMEM((C, D), jnp.float32),
        pltpu.VMEM((S,), jnp.int32),
        pltpu.VMEM((NLAB,), jnp.int32),
        pltpu.VMEM((KC,), jnp.int32),
        pltpu.VMEM((KC,), jnp.int32),
        pltpu.SemaphoreType.DMA,
        pltpu.SemaphoreType.DMA,
        pltpu.SemaphoreType.DMA,
        pltpu.SemaphoreType.DMA,
    ],
)
def _sc_embed(ids_hbm, wte, uemb, iemb, cen01, lab01,
              emb_o, nl_o, an_o,
              ids_v, nl_v, r0, r1, r2, p0, p1, p2,
              rid_stage_a, rid_stage_b, pos_stage_a, pos_stage_b,
              rows_a, rows_b, ids_row, lab_v, pres_v, dest_stage,
              sem_ga, sem_gb, sem_sa, sem_sb):
    wid = lax.axis_index("s") * NC + lax.axis_index("c")
    base = wid * TPW
    iota = lax.iota(jnp.int32, L)

    pltpu.sync_copy(ids_hbm.at[pl.ds(base, TPW)], ids_v)

    zi = jnp.zeros((L,), jnp.int32)

    def build_body(k, carry):
        ids = ids_v[pl.ds(k * L, L)]
        is_voc = ids < V
        is_usr = jnp.logical_and(ids >= V, ids < V + NU)
        is_itm = ids >= V + NU
        nl_v[pl.ds(k * L, L)] = jnp.where(is_voc, 0, 1).astype(jnp.int32)
        pos = base + k * L + iota
        rows = (ids, ids - V, ids - V - NU)
        masks = (is_voc, is_usr, is_itm)
        rlists = (r0, r1, r2)
        plists = (p0, p1, p2)
        out = []
        for t in range(3):
            mi = jnp.where(masks[t], 1, 0)
            cs = plsc.cumsum(mi)
            dest = carry[t] + cs - 1
            plsc.store_scatter(rlists[t], [dest], rows[t], mask=masks[t])
            plsc.store_scatter(plists[t], [dest], pos, mask=masks[t])
            out.append(carry[t] + plsc.all_reduce_population_count(masks[t]))
        return tuple(out)

    zv = jnp.zeros((L,), jnp.int32)
    cntv = lax.fori_loop(0, TPW // L, build_body, (zv, zv, zv))
    cnts = tuple(jnp.max(c, axis=0) for c in cntv)

    pltpu.sync_copy(nl_v, nl_o.at[pl.ds(base, TPW)])

    for t in range(3):
        cnt = cnts[t]
        rlist = (r0, r1, r2)[t]
        plist = (p0, p1, p2)[t]
        rspl = plsc.load_gather(rlist, [zi])
        pspl = plsc.load_gather(plist, [zi])
        lim = ((cnt + C - 1) // C) * C

        def fill_body(k, _, rlist=rlist, plist=plist, rspl=rspl, pspl=pspl,
                      cnt=cnt, lim=lim):
            idx = k * L + iota
            m = jnp.logical_and(idx >= cnt, idx < lim)
            rv = rlist[pl.ds(k * L, L)]
            pv = plist[pl.ds(k * L, L)]
            rlist[pl.ds(k * L, L)] = jnp.where(m, rspl, rv)
            plist[pl.ds(k * L, L)] = jnp.where(m, pspl, pv)
            return 0

        lax.fori_loop(0, LLEN // L, fill_body, 0)

    def _drain(sem):
        pltpu.make_async_copy(emb_o.at[pl.ds(0, C)], rows_a, sem).wait()

    for t, table in enumerate((wte, uemb, iemb)):
        cnt = cnts[t]
        nch = (cnt + C - 1) // C
        rlist = (r0, r1, r2)[t]
        plist = (p0, p1, p2)[t]

        def pair_body(g, _, rlist=rlist, plist=plist, table=table, cnt=cnt):
            for par in range(2):
                buf = (rows_a, rows_b)[par]
                obuf = (rows_a, rows_b)[1 - par]
                ridst = (rid_stage_a, rid_stage_b)[par]
                pstage = (pos_stage_a, pos_stage_b)[par]
                opstage = (pos_stage_a, pos_stage_b)[1 - par]
                sem_s = (sem_sa, sem_sb)[par]
                sem_g = (sem_ga, sem_gb)[par]
                osem_g = (sem_ga, sem_gb)[1 - par]
                osem_s = (sem_sa, sem_sb)[1 - par]
                ch = g * 2 + par

                @pl.when(ch * C < cnt)
                def _(ch=ch, buf=buf, obuf=obuf, ridst=ridst, pstage=pstage,
                      opstage=opstage, sem_s=sem_s, sem_g=sem_g,
                      osem_g=osem_g, osem_s=osem_s):
                    @pl.when(ch > 1)
                    def _():
                        _drain(sem_s)

                    for j in range(C // L):
                        ridst[pl.ds(j * L, L)] = rlist[pl.ds(ch * C + j * L, L)]
                        pstage[pl.ds(j * L, L)] = plist[pl.ds(ch * C + j * L, L)]
                    pltpu.async_copy(table.at[ridst], buf, sem_g)

                    @pl.when(ch > 0)
                    def _():
                        _drain(osem_g)
                        pltpu.async_copy(obuf, emb_o.at[opstage], osem_s)
            return 0

        lax.fori_loop(0, (NCH + 1) // 2, pair_body, 0)

        for par in range(2):
            buf = (rows_a, rows_b)[par]
            pstage = (pos_stage_a, pos_stage_b)[par]
            sem_g = (sem_ga, sem_gb)[par]
            sem_s = (sem_sa, sem_sb)[par]

            @pl.when(jnp.logical_and(cnt > 0, (nch - 1) % 2 == par))
            def _(buf=buf, pstage=pstage, sem_g=sem_g, sem_s=sem_s):
                _drain(sem_g)
                pltpu.async_copy(buf, emb_o.at[pstage], sem_s)

        @pl.when(cnt > 0)
        def _():
            _drain(sem_sa)

        @pl.when(cnt > C)
        def _():
            _drain(sem_sb)

    @pl.when(wid < B * 2)
    def _anode():
        b = wid // 2
        lsel = wid - 2 * b
        pltpu.sync_copy(ids_hbm.at[pl.ds(b * S, S)], ids_row)
        pltpu.sync_copy(lab01.at[pl.ds(lsel * NLAB, NLAB)], lab_v)

        pres_v[...] = jnp.zeros((KC,), jnp.int32)
        ones = jnp.ones((L,), jnp.int32)

        def pres_body(k, _):
            ids = ids_row[pl.ds(k * L, L)]
            m = ids >= V
            seq = jnp.where(m, ids - V, 0)
            lab = plsc.load_gather(lab_v, [seq])
            plsc.store_scatter(pres_v, [lab], ones, mask=m)
            return 0

        lax.fori_loop(0, S // L, pres_body, 0)

        pi = pres_v[...]
        rank = plsc.cumsum(pi) - pi
        abase = b * ANB + lsel * KC
        dest_stage[...] = jnp.where(pi > 0, abase + rank, DUMP_A + iota)

        zf = jnp.zeros((L,), jnp.float32)

        def zero_body(r, _):
            for cidx in range(D // L):
                rows_a[r, pl.ds(cidx * L, L)] = zf
            return 0

        lax.fori_loop(0, KC, zero_body, 0)
        pltpu.sync_copy(rows_a.at[pl.ds(0, KC)], an_o.at[pl.ds(abase, KC)])

        crows = rows_b.at[pl.ds(0, KC)]
        pltpu.sync_copy(cen01.at[pl.ds(lsel * KC, KC)], crows)
        pltpu.async_copy(crows, an_o.at[dest_stage], sem_ga).wait()


def kernel(input_ids, wte, user_emb, item_emb, a_centers0, a_centers1,
           a_labels0, a_labels1):
    ids = input_ids.reshape(-1).astype(jnp.int32)
    cen01 = jnp.concatenate([a_centers0, a_centers1], axis=0)
    lab01 = jnp.concatenate(
        [a_labels0.astype(jnp.int32), a_labels1.astype(jnp.int32)], axis=0)
    emb, nl, an = _sc_embed(ids, wte, user_emb, item_emb, cen01, lab01)
    return (emb.reshape(B, S, D),
            nl.reshape(B, S),
            an[:DUMP_A].reshape(B, ANB, D))
